# baseline (speedup 1.0000x reference)
import functools

import jax
import jax.numpy as jnp
import numpy as np
from jax import lax
from jax.experimental import pallas as pl
from jax.experimental.pallas import tpu as pltpu
from jax.experimental.pallas import tpu_sc as plsc

_B = 16384
_F = 2 * _B
_D = 32
_NC, _NS = 2, 16
_NW = _NC * _NS
_PER_W = _B // _NW
_CHUNK = 128
_NCHUNK = _PER_W // _CHUNK
_N = 1000000
_EROWS = _N // 2
_BROWS = _N // 64


def _eps():
  nk = jax.random.key(42)
  eps_b = jax.random.normal(
      jax.random.fold_in(nk, 0), (1, _F), dtype=jnp.float32).reshape(_B, 2)
  eps_e = jax.random.normal(
      jax.random.fold_in(nk, 1), (1, _F, _D), dtype=jnp.float32
  ).reshape(_B, 2 * _D)
  return eps_b, eps_e


def _sc_gather(eu_idx, ei_idx, bu_idx, bi_idx, bias128, ent128):
  mesh = plsc.VectorSubcoreMesh(core_axis_name="c", subcore_axis_name="s")

  @functools.partial(
      pl.kernel,
      mesh=mesh,
      out_type=[
          jax.ShapeDtypeStruct((_B, 128), jnp.float32),
          jax.ShapeDtypeStruct((_B, 128), jnp.float32),
          jax.ShapeDtypeStruct((_B, 128), jnp.float32),
          jax.ShapeDtypeStruct((_B, 128), jnp.float32),
      ],
      scratch_types=[
          pltpu.VMEM((_NCHUNK, _CHUNK), jnp.int32),
          pltpu.VMEM((_NCHUNK, _CHUNK), jnp.int32),
          pltpu.VMEM((_NCHUNK, _CHUNK), jnp.int32),
          pltpu.VMEM((_NCHUNK, _CHUNK), jnp.int32),
          pltpu.VMEM((_CHUNK, 128), jnp.float32),
          pltpu.VMEM((_CHUNK, 128), jnp.float32),
          pltpu.VMEM((_CHUNK, 128), jnp.float32),
          pltpu.VMEM((_CHUNK, 128), jnp.float32),
          pltpu.SemaphoreType.DMA,
      ],
  )
  def gather(eu_hbm, ei_hbm, bu_hbm, bi_hbm, bias_hbm, ent_hbm,
             eu_out, ei_out, bu_out, bi_out,
             eu_i, ei_i, bu_i, bi_i, eu_v, ei_v, bu_v, bi_v, sem):
    wid = lax.axis_index("s") * _NC + lax.axis_index("c")
    base = wid * _PER_W
    pltpu.sync_copy(eu_hbm.at[pl.ds(wid * _NCHUNK, _NCHUNK)], eu_i)
    pltpu.sync_copy(ei_hbm.at[pl.ds(wid * _NCHUNK, _NCHUNK)], ei_i)
    pltpu.sync_copy(bu_hbm.at[pl.ds(wid * _NCHUNK, _NCHUNK)], bu_i)
    pltpu.sync_copy(bi_hbm.at[pl.ds(wid * _NCHUNK, _NCHUNK)], bi_i)
    for c in range(_NCHUNK):
      row = base + c * _CHUNK
      w1 = pltpu.async_copy(ent_hbm.at[eu_i.at[c]], eu_v, sem)
      w2 = pltpu.async_copy(ent_hbm.at[ei_i.at[c]], ei_v, sem)
      w3 = pltpu.async_copy(bias_hbm.at[bu_i.at[c]], bu_v, sem)
      w4 = pltpu.async_copy(bias_hbm.at[bi_i.at[c]], bi_v, sem)
      w1.wait(); w2.wait(); w3.wait(); w4.wait()
      pltpu.sync_copy(eu_v, eu_out.at[pl.ds(row, _CHUNK)])
      pltpu.sync_copy(ei_v, ei_out.at[pl.ds(row, _CHUNK)])
      pltpu.sync_copy(bu_v, bu_out.at[pl.ds(row, _CHUNK)])
      pltpu.sync_copy(bi_v, bi_out.at[pl.ds(row, _CHUNK)])

  return gather(eu_idx, ei_idx, bu_idx, bi_idx, bias128, ent128)


def _softplus(v):
  return jnp.logaddexp(v, 0.0)


_RB = 1024


def _tc_body(x_ref, eu_ref, ei_ref, bu_ref, bi_ref, epsb_ref, epse_ref,
             alpha_ref, mean_ref, std_ref, klu_ref, klv_ref):
  xb = x_ref[...]
  eu2 = eu_ref[...]
  ei2 = ei_ref[...]
  bu2 = bu_ref[...]
  bi2 = bi_ref[...]
  eb = epsb_ref[...]
  ee = epse_ref[...]

  pu = (xb[:, 0:1] & 1) == 1
  pv = (xb[:, 1:2] & 1) == 1
  loc_eu = jnp.where(pu, eu2[:, 64:96], eu2[:, 0:32])
  raw_eu = jnp.where(pu, eu2[:, 96:128], eu2[:, 32:64])
  loc_ev = jnp.where(pv, ei2[:, 64:96], ei2[:, 0:32])
  raw_ev = jnp.where(pv, ei2[:, 96:128], ei2[:, 32:64])

  lanes = lax.broadcasted_iota(jnp.int32, (_RB, 128), 1)
  zero = jnp.zeros_like(bu2)

  def pick(b2, target):
    return jnp.sum(jnp.where(lanes == target, b2, zero), axis=1,
                   keepdims=True)

  mu = (xb[:, 0:1] & 63) * 2
  mv = (xb[:, 1:2] & 63) * 2
  loc_u = pick(bu2, mu)
  raw_u = pick(bu2, mu + 1)
  loc_v = pick(bi2, mv)
  raw_v = pick(bi2, mv + 1)

  sp_u = _softplus(raw_u)
  sp_v = _softplus(raw_v)
  bias_part = loc_u + loc_v + sp_u * eb[:, 0:1] + sp_v * eb[:, 1:2]

  s_u = loc_eu + _softplus(raw_eu) * ee[:, 0:_D]
  s_v = loc_ev + _softplus(raw_ev) * ee[:, _D:]
  emb = jnp.sum(s_u * s_v, axis=1, keepdims=True)

  mean_ref[...] = bias_part + emb
  klu_ref[...] = -jnp.log(sp_u) + (sp_u * sp_u + loc_u * loc_u) * 0.5 - 0.5
  klv_ref[...] = -jnp.log(sp_v) + (sp_v * sp_v + loc_v * loc_v) * 0.5 - 0.5

  @pl.when(pl.program_id(0) == 0)
  def _():
    std_ref[...] = jnp.sqrt(1.0 / _softplus(alpha_ref[...]))


def _tc_compute(x, eu, ei, bu, bi, epsb, epse, alpha11):
  grid = _B // _RB
  return pl.pallas_call(
      _tc_body,
      grid=(grid,),
      in_specs=[
          pl.BlockSpec((_RB, 2), lambda i: (i, 0)),
          pl.BlockSpec((_RB, 128), lambda i: (i, 0)),
          pl.BlockSpec((_RB, 128), lambda i: (i, 0)),
          pl.BlockSpec((_RB, 128), lambda i: (i, 0)),
          pl.BlockSpec((_RB, 128), lambda i: (i, 0)),
          pl.BlockSpec((_RB, 2), lambda i: (i, 0)),
          pl.BlockSpec((_RB, 2 * _D), lambda i: (i, 0)),
          pl.BlockSpec((1, 1), lambda i: (0, 0)),
      ],
      out_specs=[
          pl.BlockSpec((_RB, 1), lambda i: (i, 0)),
          pl.BlockSpec((1, 1), lambda i: (0, 0)),
          pl.BlockSpec((_RB, 1), lambda i: (i, 0)),
          pl.BlockSpec((_RB, 1), lambda i: (i, 0)),
      ],
      out_shape=[
          jax.ShapeDtypeStruct((_B, 1), jnp.float32),
          jax.ShapeDtypeStruct((1, 1), jnp.float32),
          jax.ShapeDtypeStruct((_B, 1), jnp.float32),
          jax.ShapeDtypeStruct((_B, 1), jnp.float32),
      ],
  )(x, eu, ei, bu, bi, epsb, epse, alpha11)


def kernel(x, bias_table, entity_table, alpha):
  xu = x[:, 0]
  xi = x[:, 1]
  shp = (_B // _CHUNK, _CHUNK)
  eu_idx = (xu >> 1).reshape(shp)
  ei_idx = (xi >> 1).reshape(shp)
  bu_idx = (xu >> 6).reshape(shp)
  bi_idx = (xi >> 6).reshape(shp)
  ent128 = entity_table.reshape(_EROWS, 128)
  bias128 = bias_table.reshape(_BROWS, 128)
  eu, ei, bu, bi = _sc_gather(
      eu_idx, ei_idx, bu_idx, bi_idx, bias128, ent128)
  eps_b, eps_e = _eps()
  mean, std, klu, klv = _tc_compute(
      x, eu, ei, bu, bi, eps_b, eps_e, alpha.reshape(1, 1))
  kl = jnp.concatenate([klu, klv], axis=1).reshape(-1)
  return (mean.reshape(-1), std.reshape(-1), kl)


# speedup vs baseline: 1.1211x; 1.1211x over previous
import functools

import jax
import jax.numpy as jnp
import numpy as np
from jax import lax
from jax.experimental import pallas as pl
from jax.experimental.pallas import tpu as pltpu
from jax.experimental.pallas import tpu_sc as plsc

_B = 16384
_F = 2 * _B
_D = 32
_NC, _NS = 2, 16
_NW = _NC * _NS
_PER_W = _B // _NW
_CHUNK = 128
_NCHUNK = _PER_W // _CHUNK
_N = 1000000
_H = _N // 2
_BROWS = _N // 64
_EK = 1000


def _ent_relayout_body(a_ref, b_ref, o_ref):
  o_ref[:, 0:64] = a_ref[...]
  o_ref[:, 64:128] = b_ref[...]


def _ent_relayout(ent):
  grid = _H // _EK
  return pl.pallas_call(
      _ent_relayout_body,
      grid=(grid,),
      in_specs=[
          pl.BlockSpec((_EK, 64), lambda i: (i, 0)),
          pl.BlockSpec((_EK, 64), lambda i: (i + _H // _EK, 0)),
      ],
      out_specs=pl.BlockSpec((_EK, 128), lambda i: (i, 0)),
      out_shape=jax.ShapeDtypeStruct((_H, 128), jnp.float32),
  )(ent, ent)


def _eps():
  nk = jax.random.key(42)
  eps_b = jax.random.normal(
      jax.random.fold_in(nk, 0), (1, _F), dtype=jnp.float32).reshape(_B, 2)
  eps_e = jax.random.normal(
      jax.random.fold_in(nk, 1), (1, _F, _D), dtype=jnp.float32
  ).reshape(_B, 2 * _D)
  return eps_b, eps_e


def _sc_gather(eu_idx, ei_idx, bu_idx, bi_idx, bias16, ent64):
  mesh = plsc.VectorSubcoreMesh(core_axis_name="c", subcore_axis_name="s")

  @functools.partial(
      pl.kernel,
      mesh=mesh,
      out_type=[
          jax.ShapeDtypeStruct((_B, 128), jnp.float32),
          jax.ShapeDtypeStruct((_B, 128), jnp.float32),
          jax.ShapeDtypeStruct((_B, 128), jnp.float32),
          jax.ShapeDtypeStruct((_B, 128), jnp.float32),
      ],
      scratch_types=[
          pltpu.VMEM((_NCHUNK, _CHUNK), jnp.int32),
          pltpu.VMEM((_NCHUNK, _CHUNK), jnp.int32),
          pltpu.VMEM((_NCHUNK, _CHUNK), jnp.int32),
          pltpu.VMEM((_NCHUNK, _CHUNK), jnp.int32),
          pltpu.VMEM((_CHUNK, 128), jnp.float32),
          pltpu.VMEM((_CHUNK, 128), jnp.float32),
          pltpu.VMEM((_CHUNK, 128), jnp.float32),
          pltpu.VMEM((_CHUNK, 128), jnp.float32),
          pltpu.SemaphoreType.DMA,
      ],
  )
  def gather(eu_hbm, ei_hbm, bu_hbm, bi_hbm, bias_hbm, ent_hbm,
             eu_out, ei_out, bu_out, bi_out,
             eu_i, ei_i, bu_i, bi_i, eu_v, ei_v, bu_v, bi_v, sem):
    wid = lax.axis_index("s") * _NC + lax.axis_index("c")
    base = wid * _PER_W
    pltpu.sync_copy(eu_hbm.at[pl.ds(wid * _NCHUNK, _NCHUNK)], eu_i)
    pltpu.sync_copy(ei_hbm.at[pl.ds(wid * _NCHUNK, _NCHUNK)], ei_i)
    pltpu.sync_copy(bu_hbm.at[pl.ds(wid * _NCHUNK, _NCHUNK)], bu_i)
    pltpu.sync_copy(bi_hbm.at[pl.ds(wid * _NCHUNK, _NCHUNK)], bi_i)
    for c in range(_NCHUNK):
      row = base + c * _CHUNK
      w1 = pltpu.async_copy(ent_hbm.at[eu_i.at[c]], eu_v, sem)
      w2 = pltpu.async_copy(ent_hbm.at[ei_i.at[c]], ei_v, sem)
      w3 = pltpu.async_copy(bias_hbm.at[bu_i.at[c]], bu_v, sem)
      w4 = pltpu.async_copy(bias_hbm.at[bi_i.at[c]], bi_v, sem)
      w1.wait(); w2.wait(); w3.wait(); w4.wait()
      pltpu.sync_copy(eu_v, eu_out.at[pl.ds(row, _CHUNK)])
      pltpu.sync_copy(ei_v, ei_out.at[pl.ds(row, _CHUNK)])
      pltpu.sync_copy(bu_v, bu_out.at[pl.ds(row, _CHUNK)])
      pltpu.sync_copy(bi_v, bi_out.at[pl.ds(row, _CHUNK)])

  return gather(eu_idx, ei_idx, bu_idx, bi_idx, bias16, ent64)


def _softplus(v):
  return jnp.logaddexp(v, 0.0)


_RB = 1024


def _tc_body(x_ref, eu_ref, ei_ref, bu_ref, bi_ref, epsb_ref, epse_ref,
             alpha_ref, mean_ref, std_ref, klu_ref, klv_ref):
  xb = x_ref[...]
  eu2 = eu_ref[...]
  ei2 = ei_ref[...]
  bu2 = bu_ref[...]
  bi2 = bi_ref[...]
  eb = epsb_ref[...]
  ee = epse_ref[...]

  pu = xb[:, 0:1] >= _H
  pv = xb[:, 1:2] >= _H
  loc_eu = jnp.where(pu, eu2[:, 64:96], eu2[:, 0:32])
  raw_eu = jnp.where(pu, eu2[:, 96:128], eu2[:, 32:64])
  loc_ev = jnp.where(pv, ei2[:, 64:96], ei2[:, 0:32])
  raw_ev = jnp.where(pv, ei2[:, 96:128], ei2[:, 32:64])

  lanes = lax.broadcasted_iota(jnp.int32, (_RB, 128), 1)
  zero = jnp.zeros_like(bu2)

  def pick(b2, target):
    return jnp.sum(jnp.where(lanes == target, b2, zero), axis=1,
                   keepdims=True)

  mu = (xb[:, 0:1] & 63) * 2
  mv = (xb[:, 1:2] & 63) * 2
  loc_u = pick(bu2, mu)
  raw_u = pick(bu2, mu + 1)
  loc_v = pick(bi2, mv)
  raw_v = pick(bi2, mv + 1)

  sp_u = _softplus(raw_u)
  sp_v = _softplus(raw_v)
  bias_part = loc_u + loc_v + sp_u * eb[:, 0:1] + sp_v * eb[:, 1:2]

  s_u = loc_eu + _softplus(raw_eu) * ee[:, 0:_D]
  s_v = loc_ev + _softplus(raw_ev) * ee[:, _D:]
  emb = jnp.sum(s_u * s_v, axis=1, keepdims=True)

  mean_ref[...] = bias_part + emb
  klu_ref[...] = -jnp.log(sp_u) + (sp_u * sp_u + loc_u * loc_u) * 0.5 - 0.5
  klv_ref[...] = -jnp.log(sp_v) + (sp_v * sp_v + loc_v * loc_v) * 0.5 - 0.5

  @pl.when(pl.program_id(0) == 0)
  def _():
    std_ref[...] = jnp.sqrt(1.0 / _softplus(alpha_ref[...]))


def _tc_compute(x, eu, ei, bu, bi, epsb, epse, alpha11):
  grid = _B // _RB
  return pl.pallas_call(
      _tc_body,
      grid=(grid,),
      in_specs=[
          pl.BlockSpec((_RB, 2), lambda i: (i, 0)),
          pl.BlockSpec((_RB, 128), lambda i: (i, 0)),
          pl.BlockSpec((_RB, 128), lambda i: (i, 0)),
          pl.BlockSpec((_RB, 128), lambda i: (i, 0)),
          pl.BlockSpec((_RB, 128), lambda i: (i, 0)),
          pl.BlockSpec((_RB, 2), lambda i: (i, 0)),
          pl.BlockSpec((_RB, 2 * _D), lambda i: (i, 0)),
          pl.BlockSpec((1, 1), lambda i: (0, 0)),
      ],
      out_specs=[
          pl.BlockSpec((_RB, 1), lambda i: (i, 0)),
          pl.BlockSpec((1, 1), lambda i: (0, 0)),
          pl.BlockSpec((_RB, 1), lambda i: (i, 0)),
          pl.BlockSpec((_RB, 1), lambda i: (i, 0)),
      ],
      out_shape=[
          jax.ShapeDtypeStruct((_B, 1), jnp.float32),
          jax.ShapeDtypeStruct((1, 1), jnp.float32),
          jax.ShapeDtypeStruct((_B, 1), jnp.float32),
          jax.ShapeDtypeStruct((_B, 1), jnp.float32),
      ],
  )(x, eu, ei, bu, bi, epsb, epse, alpha11)


def kernel(x, bias_table, entity_table, alpha):
  xu = x[:, 0]
  xi = x[:, 1]
  shp = (_B // _CHUNK, _CHUNK)
  eu_idx = jnp.where(xu >= _H, xu - _H, xu).reshape(shp)
  ei_idx = jnp.where(xi >= _H, xi - _H, xi).reshape(shp)
  bu_idx = (xu >> 6).reshape(shp)
  bi_idx = (xi >> 6).reshape(shp)
  ent128 = _ent_relayout(entity_table)
  bias128 = bias_table.reshape(_BROWS, 128)
  eu, ei, bu, bi = _sc_gather(
      eu_idx, ei_idx, bu_idx, bi_idx, bias128, ent128)
  eps_b, eps_e = _eps()
  mean, std, klu, klv = _tc_compute(
      x, eu, ei, bu, bi, eps_b, eps_e, alpha.reshape(1, 1))
  kl = jnp.concatenate([klu, klv], axis=1).reshape(-1)
  return (mean.reshape(-1), std.reshape(-1), kl)
